# baseline (speedup 1.0000x reference)
import math

import jax
import jax.numpy as jnp
from jax import lax
from jax.experimental import pallas as pl
from jax.experimental.pallas import tpu as pltpu
from jax.experimental.pallas import tpu_sc as plsc

F32 = jnp.float32

_N = 10000
_E = 320000
_D = 128

_T = 3

_NC, _NS = 2, 16
_NW = _NC * _NS
_K = 128
_NB = 4
_CHS = ((_E + _NS * _K - 1) // (_NS * _K) + _NB - 1) // _NB * _NB
_EPAD = _NS * _K * _CHS
_NPAD = 10240
_RPT = _NPAD // _NS
_ROWBLK = 400

_DH = _D // _NC
_CW = 16
_RW = _DH + _CW


def _prologue_body(x_ref, w_ref, out_ref):
    xb = x_ref[...]
    lane = lax.broadcasted_iota(jnp.int32, (xb.shape[0], _CW), 1)
    outs = []
    for c in range(_NC):
        halves = []
        for t in range(_T):
            m = lax.dot_general(xb, w_ref[t], (((1,), (0,)), ((), ())),
                                preferred_element_type=F32)
            if c == 0:
                oh = jnp.where(lane == t, 1.0, 0.0).astype(F32)
            else:
                oh = jnp.zeros((xb.shape[0], _CW), F32)
            halves.append(jnp.concatenate([m[:, c * _DH:(c + 1) * _DH], oh],
                                          axis=1))
        outs.append(jnp.stack(halves, axis=0))
    out_ref[...] = jnp.stack(outs, axis=0)


def _build_table(x, w_src):
    grid = _N // _ROWBLK
    return pl.pallas_call(
        _prologue_body,
        grid=(grid,),
        in_specs=[
            pl.BlockSpec((_ROWBLK, _D), lambda i: (i, 0)),
            pl.BlockSpec((_T, _D, _D), lambda i: (0, 0, 0)),
        ],
        out_specs=pl.BlockSpec((_NC, _T, _ROWBLK, _RW), lambda i: (0, 0, i, 0)),
        out_shape=jax.ShapeDtypeStruct((_NC, _T, _N, _RW), F32),
    )(x, w_src)


def _sc_body(table_hbm, epk_hbm, zeros_hbm, out1_hbm,
             epkA, epkB,
             idxg0, idxg1, idxg2, idxg3,
             idxs0, idxs1, idxs2, idxs3,
             rows0, rows1, rows2, rows3,
             acc, seA, seB, sg0, sg1, sg2, sg3):
    cid = lax.axis_index("c")
    sid = lax.axis_index("s")

    idxg = [idxg0, idxg1, idxg2, idxg3]
    idxs = [idxs0, idxs1, idxs2, idxs3]
    rows = [rows0, rows1, rows2, rows3]
    sg = [sg0, sg1, sg2, sg3]

    rbase = pl.multiple_of(sid * _RPT, 8)
    pltpu.sync_copy(zeros_hbm, acc.at[pl.ds(rbase, _RPT)])
    plsc.subcore_barrier()

    ebase = sid * (_CHS * _K)

    def unpack(epk_v, off, ig, isr):
        for i in range(_K // 16):
            e = epk_v[pl.ds(off + i * 16, 16)]
            et = lax.bitwise_and(lax.shift_right_logical(e, 14), 3)
            gi = lax.bitwise_and(e, 0x3FFF) + (et + cid * _T) * _N
            d = lax.shift_right_logical(e, 16)
            ig[pl.ds(i * 16, 16)] = gi
            isr[pl.ds(i * 16, 16)] = d

    def eblk(bi):
        base = pl.multiple_of(ebase + bi * (_NB * _K), 8)
        return epk_hbm.at[pl.ds(base, _NB * _K)]

    pltpu.sync_copy(eblk(0), epkA)
    for k in range(_NB):
        unpack(epkA, k * _K, idxg[k], idxs[k])
        pltpu.async_copy(table_hbm.at[idxg[k]], rows[k], sg[k])
    pltpu.async_copy(eblk(1), epkB, seB)

    def block(i, carry):
        pltpu.make_async_copy(eblk(2 * i + 1), epkB, seB).wait()
        pltpu.async_copy(eblk(2 * i + 2), epkA, seA)
        for k in range(_NB):
            pltpu.make_async_copy(table_hbm.at[idxg[k]], rows[k], sg[k]).wait()
            pltpu.sync_copy(rows[k], acc.at[idxs[k]], add=True)
            unpack(epkB, k * _K, idxg[k], idxs[k])
            pltpu.async_copy(table_hbm.at[idxg[k]], rows[k], sg[k])
        pltpu.make_async_copy(eblk(2 * i + 2), epkA, seA).wait()
        pltpu.async_copy(eblk(2 * i + 3), epkB, seB)
        for k in range(_NB):
            pltpu.make_async_copy(table_hbm.at[idxg[k]], rows[k], sg[k]).wait()
            pltpu.sync_copy(rows[k], acc.at[idxs[k]], add=True)
            unpack(epkA, k * _K, idxg[k], idxs[k])
            pltpu.async_copy(table_hbm.at[idxg[k]], rows[k], sg[k])
        return carry

    nblk = _CHS // _NB
    lax.fori_loop(0, nblk // 2, block, 0)

    for k in range(_NB):
        pltpu.make_async_copy(table_hbm.at[idxg[k]], rows[k], sg[k]).wait()
    pltpu.make_async_copy(eblk(nblk + 1), epkB, seB).wait()
    plsc.subcore_barrier()

    pltpu.sync_copy(acc.at[pl.ds(rbase, _RPT)],
                    out1_hbm.at[cid, pl.ds(rbase, _RPT)])


def _sc_scatter(table2d, epk, zeros):
    mesh = plsc.VectorSubcoreMesh(core_axis_name="c", subcore_axis_name="s")
    kern = pl.kernel(
        _sc_body,
        out_type=jax.ShapeDtypeStruct((_NC, _NPAD, _RW), F32),
        mesh=mesh,
        scratch_types=[
            pltpu.VMEM((_NB * _K,), jnp.int32),
            pltpu.VMEM((_NB * _K,), jnp.int32),
            pltpu.VMEM((_K,), jnp.int32),
            pltpu.VMEM((_K,), jnp.int32),
            pltpu.VMEM((_K,), jnp.int32),
            pltpu.VMEM((_K,), jnp.int32),
            pltpu.VMEM((_K,), jnp.int32),
            pltpu.VMEM((_K,), jnp.int32),
            pltpu.VMEM((_K,), jnp.int32),
            pltpu.VMEM((_K,), jnp.int32),
            pltpu.VMEM((_K, _RW), F32),
            pltpu.VMEM((_K, _RW), F32),
            pltpu.VMEM((_K, _RW), F32),
            pltpu.VMEM((_K, _RW), F32),
            pltpu.VMEM_SHARED((_NPAD, _RW), F32),
            pltpu.SemaphoreType.DMA,
            pltpu.SemaphoreType.DMA,
            pltpu.SemaphoreType.DMA,
            pltpu.SemaphoreType.DMA,
            pltpu.SemaphoreType.DMA,
            pltpu.SemaphoreType.DMA,
        ],
        compiler_params=pltpu.CompilerParams(
            needs_layout_passes=False, use_tc_tiling_on_sc=False),
    )
    return kern(table2d, epk, zeros)


def _epilogue_body(x_ref, p_ref, wd_ref, b_ref, wu_ref, bu_ref, g_ref,
                   be_ref, out_ref):
    xb = x_ref[...]
    p = p_ref[...]
    num = jnp.concatenate([p[0, :, :_DH], p[1, :, :_DH]], axis=1)
    cnt = jnp.zeros((xb.shape[0], 1), F32)
    for t in range(_T):
        ct = p[0, :, _DH + t:_DH + t + 1]
        bt = lax.dot_general(xb, wd_ref[t], (((1,), (0,)), ((), ())),
                             preferred_element_type=F32) + b_ref[t]
        num = num + ct * bt
        cnt = cnt + ct
    agg = num / jnp.maximum(cnt, 1.0)
    h = (lax.dot_general(xb, wu_ref[:_D], (((1,), (0,)), ((), ())),
                         preferred_element_type=F32)
         + lax.dot_general(agg, wu_ref[_D:], (((1,), (0,)), ((), ())),
                           preferred_element_type=F32)
         + bu_ref[...])
    mu = jnp.mean(h, axis=1, keepdims=True)
    d = h - mu
    var = jnp.mean(d * d, axis=1, keepdims=True)
    ln = d * lax.rsqrt(var + 1e-5) * g_ref[...] + be_ref[...]
    gelu = 0.5 * ln * (1.0 + lax.erf(ln * (1.0 / math.sqrt(2.0))))
    out_ref[...] = xb + gelu


def _epilogue(x, partials, w_dst, bst, wu, bu, gamma, beta):
    grid = _N // _ROWBLK
    return pl.pallas_call(
        _epilogue_body,
        grid=(grid,),
        in_specs=[
            pl.BlockSpec((_ROWBLK, _D), lambda i: (i, 0)),
            pl.BlockSpec((_NC, _ROWBLK, _RW), lambda i: (0, i, 0)),
            pl.BlockSpec((_T, _D, _D), lambda i: (0, 0, 0)),
            pl.BlockSpec((_T, 1, _D), lambda i: (0, 0, 0)),
            pl.BlockSpec((2 * _D, _D), lambda i: (0, 0)),
            pl.BlockSpec((1, _D), lambda i: (0, 0)),
            pl.BlockSpec((1, _D), lambda i: (0, 0)),
            pl.BlockSpec((1, _D), lambda i: (0, 0)),
        ],
        out_specs=pl.BlockSpec((_ROWBLK, _D), lambda i: (i, 0)),
        out_shape=jax.ShapeDtypeStruct((_N, _D), F32),
    )(x, partials, w_dst, bst, wu, bu, gamma, beta)


@jax.jit
def kernel(x, edge_index, edge_type, W0, b0, W1, b1, W2, b2, Wu, bu, gamma,
           beta):
    src = edge_index[0].astype(jnp.int32)
    dst = edge_index[1].astype(jnp.int32)
    et = edge_type.astype(jnp.int32)

    w_src = jnp.stack([W0[:_D], W1[:_D], W2[:_D]])
    w_dst = jnp.stack([W0[_D:], W1[_D:], W2[_D:]])
    bst = jnp.stack([b0, b1, b2])[:, None, :]

    pad = _EPAD + 2 * _NB * _K - _E
    epk = src | (et << 14) | (dst << 16)
    epk = jnp.concatenate([epk, jnp.full((pad,), _N << 16, jnp.int32)])
    zeros = jnp.zeros((_RPT, _RW), F32)

    table = _build_table(x, w_src).reshape(_NC * _T * _N, _RW)
    partials = _sc_scatter(table, epk, zeros)
    return _epilogue(x, partials, w_dst, bst, Wu, bu[None, :],
                     gamma[None, :], beta[None, :])
